# baseline (speedup 1.0000x reference)
import functools

import jax
import jax.numpy as jnp
from jax import lax
from jax.experimental import pallas as pl
from jax.experimental.pallas import tpu as pltpu
from jax.experimental.pallas import tpu_sc as plsc

N, E, D, H, B, C = 10000, 320000, 128, 128, 16, 10
NP = 10240
NPAD = NP - N
EROWS = 2560
NWORK = 32
RPW = EROWS // NWORK
SPW = NP // 16
RB = 1024
GRID = NP // RB

_HI = jax.lax.Precision.HIGHEST


def _sc_degrees(src2, dst2, zcol, ones):
    mesh = plsc.VectorSubcoreMesh(core_axis_name="c", subcore_axis_name="s")

    @functools.partial(
        pl.kernel,
        out_type=[jax.ShapeDtypeStruct((2, NP), jnp.float32),
                  jax.ShapeDtypeStruct((2, NP), jnp.float32)],
        mesh=mesh,
        scratch_types=[
            pltpu.VMEM((RPW, 128), jnp.int32),
            pltpu.VMEM((RPW, 128), jnp.int32),
            pltpu.VMEM((1, 128), jnp.float32),
            pltpu.VMEM_SHARED((NP,), jnp.float32),
            pltpu.VMEM_SHARED((NP,), jnp.float32),
            pltpu.SemaphoreType.DMA,
        ],
    )
    def deg_kernel(src_h, dst_h, z_h, ones_h, os_h, od_h,
                   sidx, didx, onev, hs, hd, sem):
        cid = lax.axis_index("c")
        sid = lax.axis_index("s")
        wid = sid * 2 + cid
        pltpu.sync_copy(z_h, hs.at[pl.ds(sid * SPW, SPW)])
        pltpu.sync_copy(z_h, hd.at[pl.ds(sid * SPW, SPW)])
        pltpu.sync_copy(ones_h, onev)
        pltpu.sync_copy(src_h.at[pl.ds(wid * RPW, RPW)], sidx)
        pltpu.sync_copy(dst_h.at[pl.ds(wid * RPW, RPW)], didx)
        plsc.subcore_barrier()

        @pl.loop(0, RPW)
        def _(j):
            pltpu.async_copy(onev.at[0], hs.at[sidx.at[j]], sem, add=True)
            pltpu.async_copy(onev.at[0], hd.at[didx.at[j]], sem, add=True)

            @pl.when(j >= 4)
            def _():
                pltpu.make_async_copy(onev.at[0], hs.at[sidx.at[j]],
                                      sem).wait()
                pltpu.make_async_copy(onev.at[0], hd.at[didx.at[j]],
                                      sem).wait()

        @pl.loop(0, 4)
        def _(j):
            pltpu.make_async_copy(onev.at[0], hs.at[sidx.at[0]], sem).wait()
            pltpu.make_async_copy(onev.at[0], hd.at[didx.at[0]], sem).wait()

        plsc.subcore_barrier()
        pltpu.sync_copy(hs.at[pl.ds(sid * SPW, SPW)],
                        os_h.at[cid, pl.ds(sid * SPW, SPW)])
        pltpu.sync_copy(hd.at[pl.ds(sid * SPW, SPW)],
                        od_h.at[cid, pl.ds(sid * SPW, SPW)])

    return deg_kernel(src2, dst2, zcol, ones)


def _sc_aggregate(hh, src2, dst2, zrows):
    mesh = plsc.VectorSubcoreMesh(core_axis_name="c", subcore_axis_name="s")

    @functools.partial(
        pl.kernel,
        out_type=jax.ShapeDtypeStruct((2, NP, 128), jnp.float32),
        mesh=mesh,
        scratch_types=[
            pltpu.VMEM((RPW // 2, 128), jnp.int32),
            pltpu.VMEM((RPW // 2, 128), jnp.int32),
            pltpu.VMEM((2, 128, 128), jnp.float32),
            pltpu.VMEM_SHARED((NP, 128), jnp.float32),
            pltpu.SemaphoreType.DMA,
            pltpu.SemaphoreType.DMA,
            pltpu.SemaphoreType.DMA,
            pltpu.SemaphoreType.DMA,
        ],
    )
    def agg_kernel(hh_h, src_h, dst_h, z_h, out_h, sidx, didx, rows, acc,
                   gsem0, gsem1, ssem0, ssem1):
        cid = lax.axis_index("c")
        sid = lax.axis_index("s")
        wid = sid * 2 + cid
        hrpw = RPW // 2
        pltpu.sync_copy(z_h, acc.at[pl.ds(sid * SPW, SPW)])
        plsc.subcore_barrier()

        for phase in range(2):
            base = wid * RPW + phase * hrpw
            pltpu.sync_copy(src_h.at[pl.ds(base, hrpw)], sidx)
            pltpu.sync_copy(dst_h.at[pl.ds(base, hrpw)], didx)
            pltpu.async_copy(hh_h.at[sidx.at[0]], rows.at[0], gsem0)
            pltpu.async_copy(hh_h.at[sidx.at[1]], rows.at[1], gsem1)

            @pl.loop(0, hrpw, step=2)
            def _(j):
                pltpu.make_async_copy(hh_h.at[sidx.at[j]], rows.at[0],
                                      gsem0).wait()
                pltpu.async_copy(rows.at[0], acc.at[didx.at[j]], ssem0,
                                 add=True)
                pltpu.make_async_copy(hh_h.at[sidx.at[j + 1]], rows.at[1],
                                      gsem1).wait()
                pltpu.async_copy(rows.at[1], acc.at[didx.at[j + 1]], ssem1,
                                 add=True)
                pltpu.make_async_copy(rows.at[0], acc.at[didx.at[j]],
                                      ssem0).wait()

                @pl.when(j + 2 < hrpw)
                def _():
                    pltpu.async_copy(hh_h.at[sidx.at[j + 2]], rows.at[0],
                                     gsem0)

                pltpu.make_async_copy(rows.at[1], acc.at[didx.at[j + 1]],
                                      ssem1).wait()

                @pl.when(j + 3 < hrpw)
                def _():
                    pltpu.async_copy(hh_h.at[sidx.at[j + 3]], rows.at[1],
                                     gsem1)

        plsc.subcore_barrier()
        pltpu.sync_copy(acc.at[pl.ds(sid * SPW, SPW)],
                        out_h.at[cid, pl.ds(sid * SPW, SPW)])

    return agg_kernel(hh, src2, dst2, zrows)


def _colify(t):
    rows = lax.broadcasted_iota(jnp.int32, (RB, 128), 0)
    lanes = lax.broadcasted_iota(jnp.int32, (RB, 128), 1)
    a = jnp.zeros((RB, 128), jnp.float32)
    for s in range(8):
        a = jnp.where(rows // 128 == s,
                      jnp.broadcast_to(t[s:s + 1, :], (RB, 128)), a)
    return jnp.sum(jnp.where(rows % 128 == lanes, a, 0.0),
                   axis=1, keepdims=True)


def _inv_sqrt_deg(t):
    return jnp.where(t > 0, lax.rsqrt(jnp.maximum(t, 1.0)), 0.0)


def _ln(t, g, b):
    mu = jnp.mean(t, axis=-1, keepdims=True)
    var = jnp.mean((t - mu) ** 2, axis=-1, keepdims=True)
    return (t - mu) * lax.rsqrt(var + 1e-5) * g + b


def _mm(a, b):
    return lax.dot_general(a, b, (((1,), (0,)), ((), ())),
                           preferred_element_type=jnp.float32, precision=_HI)


def _l2n(t):
    n = jnp.sqrt(jnp.sum(t * t, axis=1, keepdims=True))
    return t / jnp.maximum(n, 1e-12)


def _scale_body(ds_ref, x_ref, o_ref):
    ns = _colify(_inv_sqrt_deg(ds_ref[0] + ds_ref[1]))
    o_ref[...] = x_ref[...] * ns


def _tc_scale(ds3, xp):
    return pl.pallas_call(
        _scale_body,
        grid=(GRID,),
        in_specs=[
            pl.BlockSpec((2, 8, 128), lambda i: (0, i, 0)),
            pl.BlockSpec((RB, 128), lambda i: (i, 0)),
        ],
        out_specs=pl.BlockSpec((RB, 128), lambda i: (i, 0)),
        out_shape=jax.ShapeDtypeStruct((NP, 128), jnp.float32),
    )(ds3, xp)


def _layer1_body(ds_ref, dd_ref, p_ref, w_ref, b_ref, g_ref, be_ref, hh_ref):
    ns = _colify(_inv_sqrt_deg(ds_ref[0] + ds_ref[1]))
    nd = _colify(_inv_sqrt_deg(dd_ref[0] + dd_ref[1]))
    agg = (p_ref[0] + p_ref[1]) * nd
    t = _mm(agg, w_ref[...]) + b_ref[...]
    h = jnp.maximum(_ln(t, g_ref[...], be_ref[...]), 0.0)
    hh_ref[...] = h * ns


def _tc_layer1(ds3, dd3, p, w, b, g, be):
    return pl.pallas_call(
        _layer1_body,
        grid=(GRID,),
        in_specs=[
            pl.BlockSpec((2, 8, 128), lambda i: (0, i, 0)),
            pl.BlockSpec((2, 8, 128), lambda i: (0, i, 0)),
            pl.BlockSpec((2, RB, 128), lambda i: (0, i, 0)),
            pl.BlockSpec((H, H), lambda i: (0, 0)),
            pl.BlockSpec((1, H), lambda i: (0, 0)),
            pl.BlockSpec((1, H), lambda i: (0, 0)),
            pl.BlockSpec((1, H), lambda i: (0, 0)),
        ],
        out_specs=pl.BlockSpec((RB, 128), lambda i: (i, 0)),
        out_shape=jax.ShapeDtypeStruct((NP, 128), jnp.float32),
    )(ds3, dd3, p, w, b, g, be)


def _l2ph_body(dd_ref, p_ref, w_ref, b_ref, g_ref, be_ref, gid_ref,
               phys_ref, wp_ref, wc1_ref, bc1_ref, g3_ref, be3_ref,
               wc2_ref, bc2_ref, g4_ref, be4_ref, wc3_ref, bc3_ref,
               o_ref, sum_ref, max_ref, cnt_ref):
    i = pl.program_id(0)

    @pl.when(i == 0)
    def _():
        sum_ref[...] = jnp.zeros_like(sum_ref)
        cnt_ref[...] = jnp.zeros_like(cnt_ref)
        max_ref[...] = jnp.full_like(max_ref, -jnp.inf)

    nd = _colify(_inv_sqrt_deg(dd_ref[0] + dd_ref[1]))
    agg = (p_ref[0] + p_ref[1]) * nd
    t = _mm(agg, w_ref[...]) + b_ref[...]
    h = jnp.maximum(_ln(t, g_ref[...], be_ref[...]), 0.0)

    gcol = _colify(gid_ref[...])
    lane16 = lax.broadcasted_iota(jnp.int32, (RB, B), 1).astype(jnp.float32)
    oh = (gcol == lane16).astype(jnp.float32)
    sum_ref[...] += lax.dot_general(oh, h, (((0,), (0,)), ((), ())),
                                    preferred_element_type=jnp.float32,
                                    precision=_HI)
    cnt_ref[...] += lax.dot_general(oh, jnp.ones_like(h),
                                    (((0,), (0,)), ((), ())),
                                    preferred_element_type=jnp.float32,
                                    precision=_HI)
    for gb in range(B):
        m = oh[:, gb:gb + 1] > 0.5
        row = jnp.max(jnp.where(m, h, -jnp.inf), axis=0, keepdims=True)
        max_ref[gb:gb + 1, :] = jnp.maximum(max_ref[gb:gb + 1, :], row)

    @pl.when(i == GRID - 1)
    def _():
        mean = sum_ref[...] / jnp.maximum(cnt_ref[...], 1.0)
        a = _l2n(mean)
        m2 = _l2n(max_ref[...])
        ph = jnp.maximum(_mm(phys_ref[...], wp_ref[...]), 0.0)
        u = (_mm(a, wc1_ref[0:H]) + _mm(m2, wc1_ref[H:2 * H])
             + _mm(ph, wc1_ref[2 * H:3 * H]) + bc1_ref[...])
        u = jnp.maximum(_ln(u, g3_ref[...], be3_ref[...]), 0.0)
        u = jnp.maximum(_ln(_mm(u, wc2_ref[...]) + bc2_ref[...],
                            g4_ref[...], be4_ref[...]), 0.0)
        o_ref[...] = _mm(u, wc3_ref[...]) + bc3_ref[...]


def _tc_l2ph(dd3, p, w, b, g, be, gid2, phys, wp, wc1, bc1, g3, be3,
             wc2, bc2, g4, be4, wc3, bc3):
    const = lambda shape: pl.BlockSpec(shape, lambda i: tuple(0 for _ in shape))
    return pl.pallas_call(
        _l2ph_body,
        grid=(GRID,),
        in_specs=[
            pl.BlockSpec((2, 8, 128), lambda i: (0, i, 0)),
            pl.BlockSpec((2, RB, 128), lambda i: (0, i, 0)),
            const((H, H)),
            const((1, H)),
            const((1, H)),
            const((1, H)),
            pl.BlockSpec((8, 128), lambda i: (i, 0)),
            const((B, 8)),
            const((8, H)),
            const((3 * H, H)),
            const((1, H)),
            const((1, H)),
            const((1, H)),
            const((H, H)),
            const((1, H)),
            const((1, H)),
            const((1, H)),
            const((H, C)),
            const((1, C)),
        ],
        out_specs=pl.BlockSpec((B, C), lambda i: (0, 0)),
        out_shape=jax.ShapeDtypeStruct((B, C), jnp.float32),
        scratch_shapes=[
            pltpu.VMEM((B, 128), jnp.float32),
            pltpu.VMEM((B, 128), jnp.float32),
            pltpu.VMEM((B, 128), jnp.float32),
        ],
    )(dd3, p, w, b, g, be, gid2, phys, wp, wc1, bc1, g3, be3,
      wc2, bc2, g4, be4, wc3, bc3)


def kernel(x, edge_index, graph_ids, phys, W1, b1, W2, b2, g1, be1, g2, be2,
           g3, be3, g4, be4, Wp, Wc1, bc1, Wc2, bc2, Wc3, bc3):
    f32 = jnp.float32
    src = edge_index[0]
    dst = edge_index[1]
    pad = (jnp.arange(EROWS * 128 - E, dtype=jnp.int32) % NPAD) + N
    src2 = jnp.concatenate([src, pad]).reshape(EROWS, 128)
    dst2 = jnp.concatenate([dst, pad]).reshape(EROWS, 128)
    xp = jnp.pad(x, ((0, NP - N), (0, 0)))
    gid2 = jnp.pad(graph_ids, (0, NP - N),
                   constant_values=B).astype(f32).reshape(NP // 128, 128)
    zcol = jnp.zeros((SPW,), f32)
    zrows = jnp.zeros((SPW, 128), f32)
    ones = jnp.ones((1, 128), f32)

    deg_s, deg_d = _sc_degrees(src2, dst2, zcol, ones)
    ds3 = deg_s.reshape(2, NP // 128, 128)
    dd3 = deg_d.reshape(2, NP // 128, 128)

    hh1 = _tc_scale(ds3, xp)
    p1 = _sc_aggregate(hh1, src2, dst2, zrows)
    hh2 = _tc_layer1(ds3, dd3, p1, W1, b1.reshape(1, H),
                     g1.reshape(1, H), be1.reshape(1, H))
    p2 = _sc_aggregate(hh2, src2, dst2, zrows)
    return _tc_l2ph(dd3, p2, W2, b2.reshape(1, H), g2.reshape(1, H),
                    be2.reshape(1, H), gid2, phys, Wp, Wc1,
                    bc1.reshape(1, H), g3.reshape(1, H), be3.reshape(1, H),
                    Wc2, bc2.reshape(1, H), g4.reshape(1, H),
                    be4.reshape(1, H), Wc3, bc3.reshape(1, C))


# speedup vs baseline: 1.0795x; 1.0795x over previous
import functools

import jax
import jax.numpy as jnp
from jax import lax
from jax.experimental import pallas as pl
from jax.experimental.pallas import tpu as pltpu
from jax.experimental.pallas import tpu_sc as plsc

N, E, D, H, B, C = 10000, 320000, 128, 128, 16, 10
NP = 10240
NPAD = NP - N
EROWS = 2560
NWORK = 32
RPW = EROWS // NWORK
SPW = NP // 16
RB = 1024
GRID = NP // RB

_HI = jax.lax.Precision.HIGHEST


def _sc_degrees(src2, dst2, zcol, ones):
    mesh = plsc.VectorSubcoreMesh(core_axis_name="c", subcore_axis_name="s")

    @functools.partial(
        pl.kernel,
        out_type=[jax.ShapeDtypeStruct((2, NP), jnp.float32),
                  jax.ShapeDtypeStruct((2, NP), jnp.float32)],
        mesh=mesh,
        scratch_types=[
            pltpu.VMEM((RPW, 128), jnp.int32),
            pltpu.VMEM((RPW, 128), jnp.int32),
            pltpu.VMEM((1, 128), jnp.float32),
            pltpu.VMEM_SHARED((NP,), jnp.float32),
            pltpu.VMEM_SHARED((NP,), jnp.float32),
            pltpu.SemaphoreType.DMA,
        ],
    )
    def deg_kernel(src_h, dst_h, z_h, ones_h, os_h, od_h,
                   sidx, didx, onev, hs, hd, sem):
        cid = lax.axis_index("c")
        sid = lax.axis_index("s")
        wid = sid * 2 + cid
        pltpu.sync_copy(z_h, hs.at[pl.ds(sid * SPW, SPW)])
        pltpu.sync_copy(z_h, hd.at[pl.ds(sid * SPW, SPW)])
        pltpu.sync_copy(ones_h, onev)
        pltpu.sync_copy(src_h.at[pl.ds(wid * RPW, RPW)], sidx)
        pltpu.sync_copy(dst_h.at[pl.ds(wid * RPW, RPW)], didx)
        plsc.subcore_barrier()

        @pl.loop(0, RPW)
        def _(j):
            pltpu.async_copy(onev.at[0], hs.at[sidx.at[j]], sem, add=True)
            pltpu.async_copy(onev.at[0], hd.at[didx.at[j]], sem, add=True)

            @pl.when(j >= 4)
            def _():
                pltpu.make_async_copy(onev.at[0], hs.at[sidx.at[j]],
                                      sem).wait()
                pltpu.make_async_copy(onev.at[0], hd.at[didx.at[j]],
                                      sem).wait()

        @pl.loop(0, 4)
        def _(j):
            pltpu.make_async_copy(onev.at[0], hs.at[sidx.at[0]], sem).wait()
            pltpu.make_async_copy(onev.at[0], hd.at[didx.at[0]], sem).wait()

        plsc.subcore_barrier()
        pltpu.sync_copy(hs.at[pl.ds(sid * SPW, SPW)],
                        os_h.at[cid, pl.ds(sid * SPW, SPW)])
        pltpu.sync_copy(hd.at[pl.ds(sid * SPW, SPW)],
                        od_h.at[cid, pl.ds(sid * SPW, SPW)])

    return deg_kernel(src2, dst2, zcol, ones)


def _sc_aggregate(hh, src2, dst2, zrows):
    mesh = plsc.VectorSubcoreMesh(core_axis_name="c", subcore_axis_name="s")

    @functools.partial(
        pl.kernel,
        out_type=jax.ShapeDtypeStruct((2, NP, 128), jnp.float32),
        mesh=mesh,
        scratch_types=[
            pltpu.VMEM((RPW // 2, 128), jnp.int32),
            pltpu.VMEM((RPW // 2, 128), jnp.int32),
            pltpu.VMEM((2, 128, 128), jnp.float32),
            pltpu.VMEM_SHARED((NP, 128), jnp.float32),
            pltpu.SemaphoreType.DMA,
            pltpu.SemaphoreType.DMA,
            pltpu.SemaphoreType.DMA,
            pltpu.SemaphoreType.DMA,
        ],
    )
    def agg_kernel(hh_h, src_h, dst_h, z_h, out_h, sidx, didx, rows, acc,
                   gsem0, gsem1, ssem0, ssem1):
        cid = lax.axis_index("c")
        sid = lax.axis_index("s")
        wid = sid * 2 + cid
        hrpw = RPW // 2
        pltpu.sync_copy(z_h, acc.at[pl.ds(sid * SPW, SPW)])
        plsc.subcore_barrier()

        for phase in range(2):
            base = wid * RPW + phase * hrpw
            pltpu.sync_copy(src_h.at[pl.ds(base, hrpw)], sidx)
            pltpu.sync_copy(dst_h.at[pl.ds(base, hrpw)], didx)
            pltpu.async_copy(hh_h.at[sidx.at[0]], rows.at[0], gsem0)

            @pl.loop(0, hrpw, step=2)
            def _(j):
                pltpu.make_async_copy(hh_h.at[sidx.at[j]], rows.at[0],
                                      gsem0).wait()
                pltpu.async_copy(hh_h.at[sidx.at[j + 1]], rows.at[1], gsem1)
                pltpu.sync_copy(rows.at[0], acc.at[didx.at[j]], add=True)
                pltpu.make_async_copy(hh_h.at[sidx.at[j + 1]], rows.at[1],
                                      gsem1).wait()

                @pl.when(j + 2 < hrpw)
                def _():
                    pltpu.async_copy(hh_h.at[sidx.at[j + 2]], rows.at[0],
                                     gsem0)

                pltpu.sync_copy(rows.at[1], acc.at[didx.at[j + 1]], add=True)

        plsc.subcore_barrier()
        pltpu.sync_copy(acc.at[pl.ds(sid * SPW, SPW)],
                        out_h.at[cid, pl.ds(sid * SPW, SPW)])

    return agg_kernel(hh, src2, dst2, zrows)


def _colify(t):
    rows = lax.broadcasted_iota(jnp.int32, (RB, 128), 0)
    lanes = lax.broadcasted_iota(jnp.int32, (RB, 128), 1)
    a = jnp.zeros((RB, 128), jnp.float32)
    for s in range(8):
        a = jnp.where(rows // 128 == s,
                      jnp.broadcast_to(t[s:s + 1, :], (RB, 128)), a)
    return jnp.sum(jnp.where(rows % 128 == lanes, a, 0.0),
                   axis=1, keepdims=True)


def _inv_sqrt_deg(t):
    return jnp.where(t > 0, lax.rsqrt(jnp.maximum(t, 1.0)), 0.0)


def _ln(t, g, b):
    mu = jnp.mean(t, axis=-1, keepdims=True)
    var = jnp.mean((t - mu) ** 2, axis=-1, keepdims=True)
    return (t - mu) * lax.rsqrt(var + 1e-5) * g + b


def _mm(a, b):
    return lax.dot_general(a, b, (((1,), (0,)), ((), ())),
                           preferred_element_type=jnp.float32, precision=_HI)


def _l2n(t):
    n = jnp.sqrt(jnp.sum(t * t, axis=1, keepdims=True))
    return t / jnp.maximum(n, 1e-12)


def _scale_body(ds_ref, x_ref, o_ref):
    ns = _colify(_inv_sqrt_deg(ds_ref[0] + ds_ref[1]))
    o_ref[...] = x_ref[...] * ns


def _tc_scale(ds3, xp):
    return pl.pallas_call(
        _scale_body,
        grid=(GRID,),
        in_specs=[
            pl.BlockSpec((2, 8, 128), lambda i: (0, i, 0)),
            pl.BlockSpec((RB, 128), lambda i: (i, 0)),
        ],
        out_specs=pl.BlockSpec((RB, 128), lambda i: (i, 0)),
        out_shape=jax.ShapeDtypeStruct((NP, 128), jnp.float32),
    )(ds3, xp)


def _layer1_body(ds_ref, dd_ref, p_ref, w_ref, b_ref, g_ref, be_ref, hh_ref):
    ns = _colify(_inv_sqrt_deg(ds_ref[0] + ds_ref[1]))
    nd = _colify(_inv_sqrt_deg(dd_ref[0] + dd_ref[1]))
    agg = (p_ref[0] + p_ref[1]) * nd
    t = _mm(agg, w_ref[...]) + b_ref[...]
    h = jnp.maximum(_ln(t, g_ref[...], be_ref[...]), 0.0)
    hh_ref[...] = h * ns


def _tc_layer1(ds3, dd3, p, w, b, g, be):
    return pl.pallas_call(
        _layer1_body,
        grid=(GRID,),
        in_specs=[
            pl.BlockSpec((2, 8, 128), lambda i: (0, i, 0)),
            pl.BlockSpec((2, 8, 128), lambda i: (0, i, 0)),
            pl.BlockSpec((2, RB, 128), lambda i: (0, i, 0)),
            pl.BlockSpec((H, H), lambda i: (0, 0)),
            pl.BlockSpec((1, H), lambda i: (0, 0)),
            pl.BlockSpec((1, H), lambda i: (0, 0)),
            pl.BlockSpec((1, H), lambda i: (0, 0)),
        ],
        out_specs=pl.BlockSpec((RB, 128), lambda i: (i, 0)),
        out_shape=jax.ShapeDtypeStruct((NP, 128), jnp.float32),
    )(ds3, dd3, p, w, b, g, be)


def _l2ph_body(dd_ref, p_ref, w_ref, b_ref, g_ref, be_ref, gid_ref,
               phys_ref, wp_ref, wc1_ref, bc1_ref, g3_ref, be3_ref,
               wc2_ref, bc2_ref, g4_ref, be4_ref, wc3_ref, bc3_ref,
               o_ref, sum_ref, max_ref, cnt_ref):
    i = pl.program_id(0)

    @pl.when(i == 0)
    def _():
        sum_ref[...] = jnp.zeros_like(sum_ref)
        cnt_ref[...] = jnp.zeros_like(cnt_ref)
        max_ref[...] = jnp.full_like(max_ref, -jnp.inf)

    nd = _colify(_inv_sqrt_deg(dd_ref[0] + dd_ref[1]))
    agg = (p_ref[0] + p_ref[1]) * nd
    t = _mm(agg, w_ref[...]) + b_ref[...]
    h = jnp.maximum(_ln(t, g_ref[...], be_ref[...]), 0.0)

    gcol = _colify(gid_ref[...])
    lane16 = lax.broadcasted_iota(jnp.int32, (RB, B), 1).astype(jnp.float32)
    oh = (gcol == lane16).astype(jnp.float32)
    sum_ref[...] += lax.dot_general(oh, h, (((0,), (0,)), ((), ())),
                                    preferred_element_type=jnp.float32,
                                    precision=_HI)
    cnt_ref[...] += lax.dot_general(oh, jnp.ones_like(h),
                                    (((0,), (0,)), ((), ())),
                                    preferred_element_type=jnp.float32,
                                    precision=_HI)
    for gb in range(B):
        m = oh[:, gb:gb + 1] > 0.5
        row = jnp.max(jnp.where(m, h, -jnp.inf), axis=0, keepdims=True)
        max_ref[gb:gb + 1, :] = jnp.maximum(max_ref[gb:gb + 1, :], row)

    @pl.when(i == GRID - 1)
    def _():
        mean = sum_ref[...] / jnp.maximum(cnt_ref[...], 1.0)
        a = _l2n(mean)
        m2 = _l2n(max_ref[...])
        ph = jnp.maximum(_mm(phys_ref[...], wp_ref[...]), 0.0)
        u = (_mm(a, wc1_ref[0:H]) + _mm(m2, wc1_ref[H:2 * H])
             + _mm(ph, wc1_ref[2 * H:3 * H]) + bc1_ref[...])
        u = jnp.maximum(_ln(u, g3_ref[...], be3_ref[...]), 0.0)
        u = jnp.maximum(_ln(_mm(u, wc2_ref[...]) + bc2_ref[...],
                            g4_ref[...], be4_ref[...]), 0.0)
        o_ref[...] = _mm(u, wc3_ref[...]) + bc3_ref[...]


def _tc_l2ph(dd3, p, w, b, g, be, gid2, phys, wp, wc1, bc1, g3, be3,
             wc2, bc2, g4, be4, wc3, bc3):
    const = lambda shape: pl.BlockSpec(shape, lambda i: tuple(0 for _ in shape))
    return pl.pallas_call(
        _l2ph_body,
        grid=(GRID,),
        in_specs=[
            pl.BlockSpec((2, 8, 128), lambda i: (0, i, 0)),
            pl.BlockSpec((2, RB, 128), lambda i: (0, i, 0)),
            const((H, H)),
            const((1, H)),
            const((1, H)),
            const((1, H)),
            pl.BlockSpec((8, 128), lambda i: (i, 0)),
            const((B, 8)),
            const((8, H)),
            const((3 * H, H)),
            const((1, H)),
            const((1, H)),
            const((1, H)),
            const((H, H)),
            const((1, H)),
            const((1, H)),
            const((1, H)),
            const((H, C)),
            const((1, C)),
        ],
        out_specs=pl.BlockSpec((B, C), lambda i: (0, 0)),
        out_shape=jax.ShapeDtypeStruct((B, C), jnp.float32),
        scratch_shapes=[
            pltpu.VMEM((B, 128), jnp.float32),
            pltpu.VMEM((B, 128), jnp.float32),
            pltpu.VMEM((B, 128), jnp.float32),
        ],
    )(dd3, p, w, b, g, be, gid2, phys, wp, wc1, bc1, g3, be3,
      wc2, bc2, g4, be4, wc3, bc3)


def kernel(x, edge_index, graph_ids, phys, W1, b1, W2, b2, g1, be1, g2, be2,
           g3, be3, g4, be4, Wp, Wc1, bc1, Wc2, bc2, Wc3, bc3):
    f32 = jnp.float32
    src = edge_index[0]
    dst = edge_index[1]
    pad = (jnp.arange(EROWS * 128 - E, dtype=jnp.int32) % NPAD) + N
    src2 = jnp.concatenate([src, pad]).reshape(EROWS, 128)
    dst2 = jnp.concatenate([dst, pad]).reshape(EROWS, 128)
    xp = jnp.pad(x, ((0, NP - N), (0, 0)))
    gid2 = jnp.pad(graph_ids, (0, NP - N),
                   constant_values=B).astype(f32).reshape(NP // 128, 128)
    zcol = jnp.zeros((SPW,), f32)
    zrows = jnp.zeros((SPW, 128), f32)
    ones = jnp.ones((1, 128), f32)

    deg_s, deg_d = _sc_degrees(src2, dst2, zcol, ones)
    ds3 = deg_s.reshape(2, NP // 128, 128)
    dd3 = deg_d.reshape(2, NP // 128, 128)

    hh1 = _tc_scale(ds3, xp)
    p1 = _sc_aggregate(hh1, src2, dst2, zrows)
    hh2 = _tc_layer1(ds3, dd3, p1, W1, b1.reshape(1, H),
                     g1.reshape(1, H), be1.reshape(1, H))
    p2 = _sc_aggregate(hh2, src2, dst2, zrows)
    return _tc_l2ph(dd3, p2, W2, b2.reshape(1, H), g2.reshape(1, H),
                    be2.reshape(1, H), gid2, phys, Wp, Wc1,
                    bc1.reshape(1, H), g3.reshape(1, H), be3.reshape(1, H),
                    Wc2, bc2.reshape(1, H), g4.reshape(1, H),
                    be4.reshape(1, H), Wc3, bc3.reshape(1, C))


# speedup vs baseline: 1.0816x; 1.0020x over previous
import functools

import jax
import jax.numpy as jnp
from jax import lax
from jax.experimental import pallas as pl
from jax.experimental.pallas import tpu as pltpu
from jax.experimental.pallas import tpu_sc as plsc

N, E, D, H, B, C = 10000, 320000, 128, 128, 16, 10
NP = 10240
NPAD = NP - N
EROWS = 2560
NWORK = 32
RPW = EROWS // NWORK
SPW = NP // 16
RB = 1024
GRID = NP // RB

_HI = jax.lax.Precision.HIGHEST


def _sc_degrees(src2, dst2, zcol, ones):
    mesh = plsc.VectorSubcoreMesh(core_axis_name="c", subcore_axis_name="s")

    @functools.partial(
        pl.kernel,
        out_type=[jax.ShapeDtypeStruct((2, NP), jnp.float32),
                  jax.ShapeDtypeStruct((2, NP), jnp.float32)],
        mesh=mesh,
        scratch_types=[
            pltpu.VMEM((RPW, 128), jnp.int32),
            pltpu.VMEM((RPW, 128), jnp.int32),
            pltpu.VMEM((1, 128), jnp.float32),
            pltpu.VMEM_SHARED((NP,), jnp.float32),
            pltpu.VMEM_SHARED((NP,), jnp.float32),
            pltpu.SemaphoreType.DMA,
        ],
    )
    def deg_kernel(src_h, dst_h, z_h, ones_h, os_h, od_h,
                   sidx, didx, onev, hs, hd, sem):
        cid = lax.axis_index("c")
        sid = lax.axis_index("s")
        wid = sid * 2 + cid
        pltpu.sync_copy(z_h, hs.at[pl.ds(sid * SPW, SPW)])
        pltpu.sync_copy(z_h, hd.at[pl.ds(sid * SPW, SPW)])
        pltpu.sync_copy(ones_h, onev)
        pltpu.sync_copy(src_h.at[pl.ds(wid * RPW, RPW)], sidx)
        pltpu.sync_copy(dst_h.at[pl.ds(wid * RPW, RPW)], didx)
        plsc.subcore_barrier()

        @pl.loop(0, RPW)
        def _(j):
            pltpu.async_copy(onev.at[0], hs.at[sidx.at[j]], sem, add=True)
            pltpu.async_copy(onev.at[0], hd.at[didx.at[j]], sem, add=True)

            @pl.when(j >= 4)
            def _():
                pltpu.make_async_copy(onev.at[0], hs.at[sidx.at[j]],
                                      sem).wait()
                pltpu.make_async_copy(onev.at[0], hd.at[didx.at[j]],
                                      sem).wait()

        @pl.loop(0, 4)
        def _(j):
            pltpu.make_async_copy(onev.at[0], hs.at[sidx.at[0]], sem).wait()
            pltpu.make_async_copy(onev.at[0], hd.at[didx.at[0]], sem).wait()

        plsc.subcore_barrier()
        pltpu.sync_copy(hs.at[pl.ds(sid * SPW, SPW)],
                        os_h.at[cid, pl.ds(sid * SPW, SPW)])
        pltpu.sync_copy(hd.at[pl.ds(sid * SPW, SPW)],
                        od_h.at[cid, pl.ds(sid * SPW, SPW)])

    return deg_kernel(src2, dst2, zcol, ones)


def _sc_aggregate(hh, src2, dst2, zrows):
    mesh = plsc.VectorSubcoreMesh(core_axis_name="c", subcore_axis_name="s")

    @functools.partial(
        pl.kernel,
        out_type=jax.ShapeDtypeStruct((2, NP, 128), jnp.float32),
        mesh=mesh,
        scratch_types=[
            pltpu.VMEM((RPW // 2, 128), jnp.int32),
            pltpu.VMEM((RPW // 2, 128), jnp.int32),
            pltpu.VMEM((2, 128, 128), jnp.float32),
            pltpu.VMEM_SHARED((NP, 128), jnp.float32),
            pltpu.SemaphoreType.DMA,
            pltpu.SemaphoreType.DMA,
            pltpu.SemaphoreType.DMA,
            pltpu.SemaphoreType.DMA,
        ],
    )
    def agg_kernel(hh_h, src_h, dst_h, z_h, out_h, sidx, didx, rows, acc,
                   gsem0, gsem1, ssem0, ssem1):
        cid = lax.axis_index("c")
        sid = lax.axis_index("s")
        wid = sid * 2 + cid
        hrpw = RPW // 2
        pltpu.sync_copy(z_h, acc.at[pl.ds(sid * SPW, SPW)])
        plsc.subcore_barrier()

        def gat(jrow, buf, sem):
            pltpu.async_copy(hh_h.at[sidx.at[jrow, pl.ds(0, 64)]],
                             rows.at[buf, pl.ds(0, 64)], sem)
            pltpu.async_copy(hh_h.at[sidx.at[jrow, pl.ds(64, 64)]],
                             rows.at[buf, pl.ds(64, 64)], sem)

        def wgat(jrow, buf, sem):
            pltpu.make_async_copy(hh_h.at[sidx.at[jrow, pl.ds(0, 64)]],
                                  rows.at[buf, pl.ds(0, 64)], sem).wait()
            pltpu.make_async_copy(hh_h.at[sidx.at[jrow, pl.ds(64, 64)]],
                                  rows.at[buf, pl.ds(64, 64)], sem).wait()

        for phase in range(2):
            base = wid * RPW + phase * hrpw
            pltpu.sync_copy(src_h.at[pl.ds(base, hrpw)], sidx)
            pltpu.sync_copy(dst_h.at[pl.ds(base, hrpw)], didx)
            gat(0, 0, gsem0)

            @pl.loop(0, hrpw, step=2)
            def _(j):
                wgat(j, 0, gsem0)
                gat(j + 1, 1, gsem1)
                pltpu.sync_copy(rows.at[0], acc.at[didx.at[j]], add=True)
                wgat(j + 1, 1, gsem1)

                @pl.when(j + 2 < hrpw)
                def _():
                    gat(j + 2, 0, gsem0)

                pltpu.sync_copy(rows.at[1], acc.at[didx.at[j + 1]], add=True)

        plsc.subcore_barrier()
        pltpu.sync_copy(acc.at[pl.ds(sid * SPW, SPW)],
                        out_h.at[cid, pl.ds(sid * SPW, SPW)])

    return agg_kernel(hh, src2, dst2, zrows)


def _colify(t):
    rows = lax.broadcasted_iota(jnp.int32, (RB, 128), 0)
    lanes = lax.broadcasted_iota(jnp.int32, (RB, 128), 1)
    a = jnp.zeros((RB, 128), jnp.float32)
    for s in range(8):
        a = jnp.where(rows // 128 == s,
                      jnp.broadcast_to(t[s:s + 1, :], (RB, 128)), a)
    return jnp.sum(jnp.where(rows % 128 == lanes, a, 0.0),
                   axis=1, keepdims=True)


def _inv_sqrt_deg(t):
    return jnp.where(t > 0, lax.rsqrt(jnp.maximum(t, 1.0)), 0.0)


def _ln(t, g, b):
    mu = jnp.mean(t, axis=-1, keepdims=True)
    var = jnp.mean((t - mu) ** 2, axis=-1, keepdims=True)
    return (t - mu) * lax.rsqrt(var + 1e-5) * g + b


def _mm(a, b):
    return lax.dot_general(a, b, (((1,), (0,)), ((), ())),
                           preferred_element_type=jnp.float32, precision=_HI)


def _l2n(t):
    n = jnp.sqrt(jnp.sum(t * t, axis=1, keepdims=True))
    return t / jnp.maximum(n, 1e-12)


def _scale_body(ds_ref, x_ref, o_ref):
    ns = _colify(_inv_sqrt_deg(ds_ref[0] + ds_ref[1]))
    o_ref[...] = x_ref[...] * ns


def _tc_scale(ds3, xp):
    return pl.pallas_call(
        _scale_body,
        grid=(GRID,),
        in_specs=[
            pl.BlockSpec((2, 8, 128), lambda i: (0, i, 0)),
            pl.BlockSpec((RB, 128), lambda i: (i, 0)),
        ],
        out_specs=pl.BlockSpec((RB, 128), lambda i: (i, 0)),
        out_shape=jax.ShapeDtypeStruct((NP, 128), jnp.float32),
    )(ds3, xp)


def _layer1_body(ds_ref, dd_ref, p_ref, w_ref, b_ref, g_ref, be_ref, hh_ref):
    ns = _colify(_inv_sqrt_deg(ds_ref[0] + ds_ref[1]))
    nd = _colify(_inv_sqrt_deg(dd_ref[0] + dd_ref[1]))
    agg = (p_ref[0] + p_ref[1]) * nd
    t = _mm(agg, w_ref[...]) + b_ref[...]
    h = jnp.maximum(_ln(t, g_ref[...], be_ref[...]), 0.0)
    hh_ref[...] = h * ns


def _tc_layer1(ds3, dd3, p, w, b, g, be):
    return pl.pallas_call(
        _layer1_body,
        grid=(GRID,),
        in_specs=[
            pl.BlockSpec((2, 8, 128), lambda i: (0, i, 0)),
            pl.BlockSpec((2, 8, 128), lambda i: (0, i, 0)),
            pl.BlockSpec((2, RB, 128), lambda i: (0, i, 0)),
            pl.BlockSpec((H, H), lambda i: (0, 0)),
            pl.BlockSpec((1, H), lambda i: (0, 0)),
            pl.BlockSpec((1, H), lambda i: (0, 0)),
            pl.BlockSpec((1, H), lambda i: (0, 0)),
        ],
        out_specs=pl.BlockSpec((RB, 128), lambda i: (i, 0)),
        out_shape=jax.ShapeDtypeStruct((NP, 128), jnp.float32),
    )(ds3, dd3, p, w, b, g, be)


def _l2ph_body(dd_ref, p_ref, w_ref, b_ref, g_ref, be_ref, gid_ref,
               phys_ref, wp_ref, wc1_ref, bc1_ref, g3_ref, be3_ref,
               wc2_ref, bc2_ref, g4_ref, be4_ref, wc3_ref, bc3_ref,
               o_ref, sum_ref, max_ref, cnt_ref):
    i = pl.program_id(0)

    @pl.when(i == 0)
    def _():
        sum_ref[...] = jnp.zeros_like(sum_ref)
        cnt_ref[...] = jnp.zeros_like(cnt_ref)
        max_ref[...] = jnp.full_like(max_ref, -jnp.inf)

    nd = _colify(_inv_sqrt_deg(dd_ref[0] + dd_ref[1]))
    agg = (p_ref[0] + p_ref[1]) * nd
    t = _mm(agg, w_ref[...]) + b_ref[...]
    h = jnp.maximum(_ln(t, g_ref[...], be_ref[...]), 0.0)

    gcol = _colify(gid_ref[...])
    lane16 = lax.broadcasted_iota(jnp.int32, (RB, B), 1).astype(jnp.float32)
    oh = (gcol == lane16).astype(jnp.float32)
    sum_ref[...] += lax.dot_general(oh, h, (((0,), (0,)), ((), ())),
                                    preferred_element_type=jnp.float32,
                                    precision=_HI)
    cnt_ref[...] += lax.dot_general(oh, jnp.ones_like(h),
                                    (((0,), (0,)), ((), ())),
                                    preferred_element_type=jnp.float32,
                                    precision=_HI)
    for gb in range(B):
        m = oh[:, gb:gb + 1] > 0.5
        row = jnp.max(jnp.where(m, h, -jnp.inf), axis=0, keepdims=True)
        max_ref[gb:gb + 1, :] = jnp.maximum(max_ref[gb:gb + 1, :], row)

    @pl.when(i == GRID - 1)
    def _():
        mean = sum_ref[...] / jnp.maximum(cnt_ref[...], 1.0)
        a = _l2n(mean)
        m2 = _l2n(max_ref[...])
        ph = jnp.maximum(_mm(phys_ref[...], wp_ref[...]), 0.0)
        u = (_mm(a, wc1_ref[0:H]) + _mm(m2, wc1_ref[H:2 * H])
             + _mm(ph, wc1_ref[2 * H:3 * H]) + bc1_ref[...])
        u = jnp.maximum(_ln(u, g3_ref[...], be3_ref[...]), 0.0)
        u = jnp.maximum(_ln(_mm(u, wc2_ref[...]) + bc2_ref[...],
                            g4_ref[...], be4_ref[...]), 0.0)
        o_ref[...] = _mm(u, wc3_ref[...]) + bc3_ref[...]


def _tc_l2ph(dd3, p, w, b, g, be, gid2, phys, wp, wc1, bc1, g3, be3,
             wc2, bc2, g4, be4, wc3, bc3):
    const = lambda shape: pl.BlockSpec(shape, lambda i: tuple(0 for _ in shape))
    return pl.pallas_call(
        _l2ph_body,
        grid=(GRID,),
        in_specs=[
            pl.BlockSpec((2, 8, 128), lambda i: (0, i, 0)),
            pl.BlockSpec((2, RB, 128), lambda i: (0, i, 0)),
            const((H, H)),
            const((1, H)),
            const((1, H)),
            const((1, H)),
            pl.BlockSpec((8, 128), lambda i: (i, 0)),
            const((B, 8)),
            const((8, H)),
            const((3 * H, H)),
            const((1, H)),
            const((1, H)),
            const((1, H)),
            const((H, H)),
            const((1, H)),
            const((1, H)),
            const((1, H)),
            const((H, C)),
            const((1, C)),
        ],
        out_specs=pl.BlockSpec((B, C), lambda i: (0, 0)),
        out_shape=jax.ShapeDtypeStruct((B, C), jnp.float32),
        scratch_shapes=[
            pltpu.VMEM((B, 128), jnp.float32),
            pltpu.VMEM((B, 128), jnp.float32),
            pltpu.VMEM((B, 128), jnp.float32),
        ],
    )(dd3, p, w, b, g, be, gid2, phys, wp, wc1, bc1, g3, be3,
      wc2, bc2, g4, be4, wc3, bc3)


def kernel(x, edge_index, graph_ids, phys, W1, b1, W2, b2, g1, be1, g2, be2,
           g3, be3, g4, be4, Wp, Wc1, bc1, Wc2, bc2, Wc3, bc3):
    f32 = jnp.float32
    src = edge_index[0]
    dst = edge_index[1]
    pad = (jnp.arange(EROWS * 128 - E, dtype=jnp.int32) % NPAD) + N
    src2 = jnp.concatenate([src, pad]).reshape(EROWS, 128)
    dst2 = jnp.concatenate([dst, pad]).reshape(EROWS, 128)
    xp = jnp.pad(x, ((0, NP - N), (0, 0)))
    gid2 = jnp.pad(graph_ids, (0, NP - N),
                   constant_values=B).astype(f32).reshape(NP // 128, 128)
    zcol = jnp.zeros((SPW,), f32)
    zrows = jnp.zeros((SPW, 128), f32)
    ones = jnp.ones((1, 128), f32)

    deg_s, deg_d = _sc_degrees(src2, dst2, zcol, ones)
    ds3 = deg_s.reshape(2, NP // 128, 128)
    dd3 = deg_d.reshape(2, NP // 128, 128)

    hh1 = _tc_scale(ds3, xp)
    p1 = _sc_aggregate(hh1, src2, dst2, zrows)
    hh2 = _tc_layer1(ds3, dd3, p1, W1, b1.reshape(1, H),
                     g1.reshape(1, H), be1.reshape(1, H))
    p2 = _sc_aggregate(hh2, src2, dst2, zrows)
    return _tc_l2ph(dd3, p2, W2, b2.reshape(1, H), g2.reshape(1, H),
                    be2.reshape(1, H), gid2, phys, Wp, Wc1,
                    bc1.reshape(1, H), g3.reshape(1, H), be3.reshape(1, H),
                    Wc2, bc2.reshape(1, H), g4.reshape(1, H),
                    be4.reshape(1, H), Wc3, bc3.reshape(1, C))


# speedup vs baseline: 1.0851x; 1.0032x over previous
import functools

import jax
import jax.numpy as jnp
from jax import lax
from jax.experimental import pallas as pl
from jax.experimental.pallas import tpu as pltpu
from jax.experimental.pallas import tpu_sc as plsc

N, E, D, H, B, C = 10000, 320000, 128, 128, 16, 10
NP = 10240
NPAD = NP - N
EROWS = 2560
NWORK = 32
RPW = EROWS // NWORK
SPW = NP // 16
RB = 1024
GRID = NP // RB

_HI = jax.lax.Precision.HIGHEST


def _sc_degrees(src2, dst2, zcol, ones):
    mesh = plsc.VectorSubcoreMesh(core_axis_name="c", subcore_axis_name="s")

    @functools.partial(
        pl.kernel,
        out_type=[jax.ShapeDtypeStruct((2, NP), jnp.float32),
                  jax.ShapeDtypeStruct((2, NP), jnp.float32)],
        mesh=mesh,
        scratch_types=[
            pltpu.VMEM((RPW, 128), jnp.int32),
            pltpu.VMEM((RPW, 128), jnp.int32),
            pltpu.VMEM((1, 128), jnp.float32),
            pltpu.VMEM_SHARED((NP,), jnp.float32),
            pltpu.VMEM_SHARED((NP,), jnp.float32),
            pltpu.SemaphoreType.DMA,
        ],
    )
    def deg_kernel(src_h, dst_h, z_h, ones_h, os_h, od_h,
                   sidx, didx, onev, hs, hd, sem):
        cid = lax.axis_index("c")
        sid = lax.axis_index("s")
        wid = sid * 2 + cid
        pltpu.sync_copy(z_h, hs.at[pl.ds(sid * SPW, SPW)])
        pltpu.sync_copy(z_h, hd.at[pl.ds(sid * SPW, SPW)])
        pltpu.sync_copy(ones_h, onev)
        pltpu.sync_copy(src_h.at[pl.ds(wid * RPW, RPW)], sidx)
        pltpu.sync_copy(dst_h.at[pl.ds(wid * RPW, RPW)], didx)
        plsc.subcore_barrier()

        @pl.loop(0, RPW)
        def _(j):
            pltpu.async_copy(onev.at[0], hs.at[sidx.at[j]], sem, add=True)
            pltpu.async_copy(onev.at[0], hd.at[didx.at[j]], sem, add=True)

            @pl.when(j >= 8)
            def _():
                pltpu.make_async_copy(onev.at[0], hs.at[sidx.at[0]],
                                      sem).wait()
                pltpu.make_async_copy(onev.at[0], hd.at[didx.at[0]],
                                      sem).wait()

        @pl.loop(0, 8)
        def _(j):
            pltpu.make_async_copy(onev.at[0], hs.at[sidx.at[0]], sem).wait()
            pltpu.make_async_copy(onev.at[0], hd.at[didx.at[0]], sem).wait()

        plsc.subcore_barrier()
        pltpu.sync_copy(hs.at[pl.ds(sid * SPW, SPW)],
                        os_h.at[cid, pl.ds(sid * SPW, SPW)])
        pltpu.sync_copy(hd.at[pl.ds(sid * SPW, SPW)],
                        od_h.at[cid, pl.ds(sid * SPW, SPW)])

    return deg_kernel(src2, dst2, zcol, ones)


def _sc_aggregate(hh, src2, dst2, zrows):
    mesh = plsc.VectorSubcoreMesh(core_axis_name="c", subcore_axis_name="s")

    @functools.partial(
        pl.kernel,
        out_type=jax.ShapeDtypeStruct((2, NP, 128), jnp.float32),
        mesh=mesh,
        scratch_types=[
            pltpu.VMEM((RPW // 2, 128), jnp.int32),
            pltpu.VMEM((RPW // 2, 128), jnp.int32),
            pltpu.VMEM((2, 128, 128), jnp.float32),
            pltpu.VMEM_SHARED((NP, 128), jnp.float32),
            pltpu.SemaphoreType.DMA,
            pltpu.SemaphoreType.DMA,
            pltpu.SemaphoreType.DMA,
            pltpu.SemaphoreType.DMA,
        ],
    )
    def agg_kernel(hh_h, src_h, dst_h, z_h, out_h, sidx, didx, rows, acc,
                   gsem0, gsem1, ssem0, ssem1):
        cid = lax.axis_index("c")
        sid = lax.axis_index("s")
        wid = sid * 2 + cid
        hrpw = RPW // 2
        pltpu.sync_copy(z_h, rows.at[0])
        for zi in range(SPW // 128):
            pltpu.sync_copy(rows.at[0],
                            acc.at[pl.ds(sid * SPW + zi * 128, 128)])
        plsc.subcore_barrier()

        def gat(jrow, buf, sem):
            pltpu.async_copy(hh_h.at[sidx.at[jrow, pl.ds(0, 64)]],
                             rows.at[buf, pl.ds(0, 64)], sem)
            pltpu.async_copy(hh_h.at[sidx.at[jrow, pl.ds(64, 64)]],
                             rows.at[buf, pl.ds(64, 64)], sem)

        def wgat(jrow, buf, sem):
            pltpu.make_async_copy(hh_h.at[sidx.at[jrow, pl.ds(0, 64)]],
                                  rows.at[buf, pl.ds(0, 64)], sem).wait()
            pltpu.make_async_copy(hh_h.at[sidx.at[jrow, pl.ds(64, 64)]],
                                  rows.at[buf, pl.ds(64, 64)], sem).wait()

        for phase in range(2):
            base = wid * RPW + phase * hrpw
            pltpu.sync_copy(src_h.at[pl.ds(base, hrpw)], sidx)
            pltpu.sync_copy(dst_h.at[pl.ds(base, hrpw)], didx)
            gat(0, 0, gsem0)

            @pl.loop(0, hrpw, step=2)
            def _(j):
                wgat(j, 0, gsem0)
                gat(j + 1, 1, gsem1)
                pltpu.sync_copy(rows.at[0], acc.at[didx.at[j]], add=True)
                wgat(j + 1, 1, gsem1)

                @pl.when(j + 2 < hrpw)
                def _():
                    gat(j + 2, 0, gsem0)

                pltpu.sync_copy(rows.at[1], acc.at[didx.at[j + 1]], add=True)

        plsc.subcore_barrier()
        pltpu.sync_copy(acc.at[pl.ds(sid * SPW, SPW)],
                        out_h.at[cid, pl.ds(sid * SPW, SPW)])

    return agg_kernel(hh, src2, dst2, zrows)


def _colify(t):
    rows = lax.broadcasted_iota(jnp.int32, (RB, 128), 0)
    lanes = lax.broadcasted_iota(jnp.int32, (RB, 128), 1)
    a = jnp.zeros((RB, 128), jnp.float32)
    for s in range(8):
        a = jnp.where(rows // 128 == s,
                      jnp.broadcast_to(t[s:s + 1, :], (RB, 128)), a)
    return jnp.sum(jnp.where(rows % 128 == lanes, a, 0.0),
                   axis=1, keepdims=True)


def _inv_sqrt_deg(t):
    return jnp.where(t > 0, lax.rsqrt(jnp.maximum(t, 1.0)), 0.0)


def _ln(t, g, b):
    mu = jnp.mean(t, axis=-1, keepdims=True)
    var = jnp.mean((t - mu) ** 2, axis=-1, keepdims=True)
    return (t - mu) * lax.rsqrt(var + 1e-5) * g + b


def _mm(a, b):
    return lax.dot_general(a, b, (((1,), (0,)), ((), ())),
                           preferred_element_type=jnp.float32, precision=_HI)


def _l2n(t):
    n = jnp.sqrt(jnp.sum(t * t, axis=1, keepdims=True))
    return t / jnp.maximum(n, 1e-12)


def _scale_body(ds_ref, x_ref, o_ref):
    ns = _colify(_inv_sqrt_deg(ds_ref[0] + ds_ref[1]))
    o_ref[...] = x_ref[...] * ns


def _tc_scale(ds3, xp):
    return pl.pallas_call(
        _scale_body,
        grid=(GRID,),
        in_specs=[
            pl.BlockSpec((2, 8, 128), lambda i: (0, i, 0)),
            pl.BlockSpec((RB, 128), lambda i: (i, 0)),
        ],
        out_specs=pl.BlockSpec((RB, 128), lambda i: (i, 0)),
        out_shape=jax.ShapeDtypeStruct((NP, 128), jnp.float32),
    )(ds3, xp)


def _layer1_body(ds_ref, dd_ref, p_ref, w_ref, b_ref, g_ref, be_ref, hh_ref):
    ns = _colify(_inv_sqrt_deg(ds_ref[0] + ds_ref[1]))
    nd = _colify(_inv_sqrt_deg(dd_ref[0] + dd_ref[1]))
    agg = (p_ref[0] + p_ref[1]) * nd
    t = _mm(agg, w_ref[...]) + b_ref[...]
    h = jnp.maximum(_ln(t, g_ref[...], be_ref[...]), 0.0)
    hh_ref[...] = h * ns


def _tc_layer1(ds3, dd3, p, w, b, g, be):
    return pl.pallas_call(
        _layer1_body,
        grid=(GRID,),
        in_specs=[
            pl.BlockSpec((2, 8, 128), lambda i: (0, i, 0)),
            pl.BlockSpec((2, 8, 128), lambda i: (0, i, 0)),
            pl.BlockSpec((2, RB, 128), lambda i: (0, i, 0)),
            pl.BlockSpec((H, H), lambda i: (0, 0)),
            pl.BlockSpec((1, H), lambda i: (0, 0)),
            pl.BlockSpec((1, H), lambda i: (0, 0)),
            pl.BlockSpec((1, H), lambda i: (0, 0)),
        ],
        out_specs=pl.BlockSpec((RB, 128), lambda i: (i, 0)),
        out_shape=jax.ShapeDtypeStruct((NP, 128), jnp.float32),
    )(ds3, dd3, p, w, b, g, be)


def _l2ph_body(dd_ref, p_ref, w_ref, b_ref, g_ref, be_ref, gid_ref,
               phys_ref, wp_ref, wc1_ref, bc1_ref, g3_ref, be3_ref,
               wc2_ref, bc2_ref, g4_ref, be4_ref, wc3_ref, bc3_ref,
               o_ref, sum_ref, max_ref, cnt_ref):
    i = pl.program_id(0)

    @pl.when(i == 0)
    def _():
        sum_ref[...] = jnp.zeros_like(sum_ref)
        cnt_ref[...] = jnp.zeros_like(cnt_ref)
        max_ref[...] = jnp.full_like(max_ref, -jnp.inf)

    nd = _colify(_inv_sqrt_deg(dd_ref[0] + dd_ref[1]))
    agg = (p_ref[0] + p_ref[1]) * nd
    t = _mm(agg, w_ref[...]) + b_ref[...]
    h = jnp.maximum(_ln(t, g_ref[...], be_ref[...]), 0.0)

    gcol = _colify(gid_ref[...])
    lane16 = lax.broadcasted_iota(jnp.int32, (RB, B), 1).astype(jnp.float32)
    oh = (gcol == lane16).astype(jnp.float32)
    sum_ref[...] += lax.dot_general(oh, h, (((0,), (0,)), ((), ())),
                                    preferred_element_type=jnp.float32,
                                    precision=_HI)
    cnt_ref[...] += lax.dot_general(oh, jnp.ones_like(h),
                                    (((0,), (0,)), ((), ())),
                                    preferred_element_type=jnp.float32,
                                    precision=_HI)
    for gb in range(B):
        m = oh[:, gb:gb + 1] > 0.5
        row = jnp.max(jnp.where(m, h, -jnp.inf), axis=0, keepdims=True)
        max_ref[gb:gb + 1, :] = jnp.maximum(max_ref[gb:gb + 1, :], row)

    @pl.when(i == GRID - 1)
    def _():
        mean = sum_ref[...] / jnp.maximum(cnt_ref[...], 1.0)
        a = _l2n(mean)
        m2 = _l2n(max_ref[...])
        ph = jnp.maximum(_mm(phys_ref[...], wp_ref[...]), 0.0)
        u = (_mm(a, wc1_ref[0:H]) + _mm(m2, wc1_ref[H:2 * H])
             + _mm(ph, wc1_ref[2 * H:3 * H]) + bc1_ref[...])
        u = jnp.maximum(_ln(u, g3_ref[...], be3_ref[...]), 0.0)
        u = jnp.maximum(_ln(_mm(u, wc2_ref[...]) + bc2_ref[...],
                            g4_ref[...], be4_ref[...]), 0.0)
        o_ref[...] = _mm(u, wc3_ref[...]) + bc3_ref[...]


def _tc_l2ph(dd3, p, w, b, g, be, gid2, phys, wp, wc1, bc1, g3, be3,
             wc2, bc2, g4, be4, wc3, bc3):
    const = lambda shape: pl.BlockSpec(shape, lambda i: tuple(0 for _ in shape))
    return pl.pallas_call(
        _l2ph_body,
        grid=(GRID,),
        in_specs=[
            pl.BlockSpec((2, 8, 128), lambda i: (0, i, 0)),
            pl.BlockSpec((2, RB, 128), lambda i: (0, i, 0)),
            const((H, H)),
            const((1, H)),
            const((1, H)),
            const((1, H)),
            pl.BlockSpec((8, 128), lambda i: (i, 0)),
            const((B, 8)),
            const((8, H)),
            const((3 * H, H)),
            const((1, H)),
            const((1, H)),
            const((1, H)),
            const((H, H)),
            const((1, H)),
            const((1, H)),
            const((1, H)),
            const((H, C)),
            const((1, C)),
        ],
        out_specs=pl.BlockSpec((B, C), lambda i: (0, 0)),
        out_shape=jax.ShapeDtypeStruct((B, C), jnp.float32),
        scratch_shapes=[
            pltpu.VMEM((B, 128), jnp.float32),
            pltpu.VMEM((B, 128), jnp.float32),
            pltpu.VMEM((B, 128), jnp.float32),
        ],
    )(dd3, p, w, b, g, be, gid2, phys, wp, wc1, bc1, g3, be3,
      wc2, bc2, g4, be4, wc3, bc3)


def kernel(x, edge_index, graph_ids, phys, W1, b1, W2, b2, g1, be1, g2, be2,
           g3, be3, g4, be4, Wp, Wc1, bc1, Wc2, bc2, Wc3, bc3):
    f32 = jnp.float32
    src = edge_index[0]
    dst = edge_index[1]
    pad = (jnp.arange(EROWS * 128 - E, dtype=jnp.int32) % NPAD) + N
    src2 = jnp.concatenate([src, pad]).reshape(EROWS, 128)
    dst2 = jnp.concatenate([dst, pad]).reshape(EROWS, 128)
    xp = jnp.pad(x, ((0, NP - N), (0, 0)))
    gid2 = jnp.pad(graph_ids, (0, NP - N),
                   constant_values=B).astype(f32).reshape(NP // 128, 128)
    zcol = jnp.zeros((SPW,), f32)
    zrows = jnp.zeros((128, 128), f32)
    ones = jnp.ones((1, 128), f32)

    deg_s, deg_d = _sc_degrees(src2, dst2, zcol, ones)
    ds3 = deg_s.reshape(2, NP // 128, 128)
    dd3 = deg_d.reshape(2, NP // 128, 128)

    hh1 = _tc_scale(ds3, xp)
    p1 = _sc_aggregate(hh1, src2, dst2, zrows)
    hh2 = _tc_layer1(ds3, dd3, p1, W1, b1.reshape(1, H),
                     g1.reshape(1, H), be1.reshape(1, H))
    p2 = _sc_aggregate(hh2, src2, dst2, zrows)
    return _tc_l2ph(dd3, p2, W2, b2.reshape(1, H), g2.reshape(1, H),
                    be2.reshape(1, H), gid2, phys, Wp, Wc1,
                    bc1.reshape(1, H), g3.reshape(1, H), be3.reshape(1, H),
                    Wc2, bc2.reshape(1, H), g4.reshape(1, H),
                    be4.reshape(1, H), Wc3, bc3.reshape(1, C))
